# baseline (speedup 1.0000x reference)
import functools

import jax
import jax.numpy as jnp
from jax import lax
from jax.experimental import pallas as pl
from jax.experimental.pallas import tpu as pltpu
from jax.experimental.pallas import tpu_sc as plsc

B = 16384
L = 200
EMB = 32
LB = 4
TBW = 128


def _build_gather():
    info = plsc.get_sparse_core_info()
    nc, ns = info.num_cores, info.num_subcores
    nw = nc * ns
    n_lb = L // LB
    n_tb = B // TBW
    tb_per_w = n_tb // nw
    n_units = n_lb * tb_per_w
    t_bytes = LB * 4 * 8 * 128 * 4

    mesh = plsc.VectorSubcoreMesh(core_axis_name="c", subcore_axis_name="s")

    @functools.partial(
        pl.kernel,
        mesh=mesh,
        compiler_params=pltpu.CompilerParams(use_tc_tiling_on_sc=False,
                                             needs_layout_passes=False),
        out_type=jax.ShapeDtypeStruct((L, 4, 128, 8, 128), jnp.float32),
        scratch_types=[
            pltpu.VMEM((2, LB, TBW), jnp.int32),
            pltpu.VMEM((2, LB, TBW, EMB), jnp.float32),
            pltpu.VMEM((2, LB, 4, 8, 128), jnp.float32),
            pltpu.SemaphoreType.DMA,
            pltpu.SemaphoreType.DMA,
            pltpu.SemaphoreType.DMA,
            pltpu.SemaphoreType.DMA,
        ],
    )
    def gather_kernel(table_hbm, idx_hbm, out_hbm, idx_v, rows_v, t_v,
                      g0, g1, w0, w1):
        g_sems = (g0, g1)
        w_sems = (w0, w1)
        wid = lax.axis_index("s") * nc + lax.axis_index("c")
        tb0 = wid * tb_per_w

        def unit_coords(u):
            lb = u // tb_per_w
            tb = tb0 + lax.rem(u, tb_per_w)
            return lb * LB, tb

        def stage_and_fire(u, b):
            l0, tb = unit_coords(u)
            pltpu.sync_copy(
                idx_hbm.at[pl.ds(l0, LB), pl.ds(tb * TBW, TBW)], idx_v.at[b])
            for d in range(LB):
                pltpu.async_copy(table_hbm.at[idx_v.at[b, d]],
                                 rows_v.at[b, d], g_sems[b])

        def wait_gathers(b):
            for d in range(LB):
                pltpu.make_async_copy(table_hbm.at[idx_v.at[b, d]],
                                      rows_v.at[b, d], g_sems[b]).wait()

        def transpose_unit(b):
            lane = lax.iota(jnp.int32, 16)

            for d in range(LB):
                rows2d = rows_v.at[b, d]
                tloc = t_v.at[b, d]

                @plsc.parallel_loop(0, EMB, unroll=4)
                def c_body(c):
                    col = lax.bitwise_and(c + lane, EMB - 1)
                    s_idx = lax.shift_right_logical(col, 3)
                    i_idx = lax.bitwise_and(col, 7)
                    for g in range(8):
                        row = g * 16 + lane
                        v = plsc.load_gather(rows2d, [row, col])
                        plsc.store_scatter(tloc, [s_idx, i_idx, row], v)

        def wait_write(b):
            pltpu.make_async_copy(
                t_v.at[b], out_hbm.at[pl.ds(0, LB), :, tb0],
                w_sems[b]).wait()

        stage_and_fire(0, 0)

        def body(u, carry):
            b = lax.rem(u, 2)

            nxt = jnp.minimum(u + 1, n_units - 1)

            @pl.when(b == 0)
            def _():
                stage_and_fire(nxt, 1)
                wait_gathers(0)

                @pl.when(u >= 2)
                def _():
                    wait_write(0)

                transpose_unit(0)
                l0, tb = unit_coords(u)
                pltpu.async_copy(
                    t_v.at[0],
                    out_hbm.at[pl.ds(l0, LB), :, tb], w0)

            @pl.when(b == 1)
            def _():
                stage_and_fire(nxt, 0)
                wait_gathers(1)

                @pl.when(u >= 2)
                def _():
                    wait_write(1)

                transpose_unit(1)
                l0, tb = unit_coords(u)
                pltpu.async_copy(
                    t_v.at[1],
                    out_hbm.at[pl.ds(l0, LB), :, tb], w1)

            return carry

        lax.fori_loop(0, n_units, body, 0)

        wait_write(0)
        wait_write(1)
        wait_gathers(0)

    return gather_kernel


def _build_table_conv():
    info = plsc.get_sparse_core_info()
    nc, ns = info.num_cores, info.num_subcores
    nw = nc * ns
    n_ct = 7813

    mesh = plsc.VectorSubcoreMesh(core_axis_name="c", subcore_axis_name="s")

    @functools.partial(
        pl.kernel,
        mesh=mesh,
        compiler_params=pltpu.CompilerParams(use_tc_tiling_on_sc=True,
                                             needs_layout_passes=False,
                                             disable_bounds_checks=True),
        out_type=jax.ShapeDtypeStruct((250016, 128), jnp.float32),
        scratch_types=[
            pltpu.VMEM((2, 32, 128), jnp.float32),
            pltpu.VMEM((2, 32, 128), jnp.float32),
            pltpu.SemaphoreType.DMA,
            pltpu.SemaphoreType.DMA,
            pltpu.SemaphoreType.DMA,
            pltpu.SemaphoreType.DMA,
        ],
    )
    def conv_kernel(tt_hbm, im_hbm, src_v, dst_v, i0, i1, o0, o1):
        i_sems = (i0, i1)
        o_sems = (o0, o1)
        wid = lax.axis_index("s") * nc + lax.axis_index("c")
        n_u = (n_ct - wid + nw - 1) // nw
        lane = lax.iota(jnp.int32, 16)

        def fetch(u, b):
            ct = wid + u * nw
            pltpu.async_copy(tt_hbm.at[:, pl.ds(ct * 128, 128)],
                             src_v.at[b], i_sems[b])

        def wait_fetch(b):
            pltpu.make_async_copy(tt_hbm.at[:, pl.ds(0, 128)],
                                  src_v.at[b], i_sems[b]).wait()

        def wait_out(b):
            pltpu.make_async_copy(dst_v.at[b], im_hbm.at[pl.ds(0, 32)],
                                  o_sems[b]).wait()

        def transpose(b):
            @plsc.parallel_loop(0, 32, unroll=2)
            def c_body(c):
                col = lax.bitwise_and(c + lane, 31)
                for g in range(8):
                    vbase = g * 16 + lane
                    v = plsc.load_gather(src_v.at[b], [col, vbase])
                    plsc.store_scatter(
                        dst_v.at[b],
                        [lax.shift_right_logical(vbase, 2),
                         lax.bitwise_and(vbase, 3) * 32 + col],
                        v)

        fetch(0, 0)

        def body(u, carry):
            b = lax.rem(u, 2)
            nxt = jnp.minimum(u + 1, n_u - 1)

            @pl.when(b == 0)
            def _():
                fetch(nxt, 1)
                wait_fetch(0)

                @pl.when(u >= 2)
                def _():
                    wait_out(0)

                transpose(0)
                ct = wid + u * nw
                pltpu.async_copy(dst_v.at[0],
                                 im_hbm.at[pl.ds(ct * 32, 32)], o0)

            @pl.when(b == 1)
            def _():
                fetch(nxt, 0)
                wait_fetch(1)

                @pl.when(u >= 2)
                def _():
                    wait_out(1)

                transpose(1)
                ct = wid + u * nw
                pltpu.async_copy(dst_v.at[1],
                                 im_hbm.at[pl.ds(ct * 32, 32)], o1)

            return carry

        lax.fori_loop(0, n_u, body, 0)
        wait_out(0)
        wait_out(1)

        @pl.when(lax.rem(n_u, 2) == 0)
        def _():
            wait_fetch(0)

        @pl.when(lax.rem(n_u, 2) == 1)
        def _():
            wait_fetch(1)

    return conv_kernel


def kernel(prev_actions, table):
    idxT = jnp.transpose(prev_actions)
    im = _build_table_conv()(jnp.transpose(table))
    w = _build_gather()(im.reshape(1000064, 32), idxT)
    out = w.transpose(0, 1, 3, 2, 4).reshape(L, EMB, B).transpose(2, 0, 1)
    return out


# speedup vs baseline: 1.1342x; 1.1342x over previous
import functools

import jax
import jax.numpy as jnp
from jax import lax
from jax.experimental import pallas as pl
from jax.experimental.pallas import tpu as pltpu
from jax.experimental.pallas import tpu_sc as plsc

B = 16384
L = 200
EMB = 32
LB = 4
TBW = 128


def _build_gather():
    info = plsc.get_sparse_core_info()
    nc, ns = info.num_cores, info.num_subcores
    nw = nc * ns
    n_lb = L // LB
    n_tb = B // TBW
    tb_per_w = n_tb // nw
    n_units = n_lb * tb_per_w
    t_bytes = LB * 4 * 8 * 128 * 4

    mesh = plsc.VectorSubcoreMesh(core_axis_name="c", subcore_axis_name="s")

    @functools.partial(
        pl.kernel,
        mesh=mesh,
        compiler_params=pltpu.CompilerParams(use_tc_tiling_on_sc=False,
                                             needs_layout_passes=False),
        out_type=jax.ShapeDtypeStruct((L, 4, 128, 8, 128), jnp.float32),
        scratch_types=[
            pltpu.VMEM((2, LB, TBW), jnp.int32),
            pltpu.VMEM((2, LB, TBW, EMB), jnp.float32),
            pltpu.VMEM((2, LB, 4, 8, 128), jnp.float32),
            pltpu.SemaphoreType.DMA,
            pltpu.SemaphoreType.DMA,
            pltpu.SemaphoreType.DMA,
            pltpu.SemaphoreType.DMA,
        ],
    )
    def gather_kernel(table_hbm, idx_hbm, out_hbm, idx_v, rows_v, t_v,
                      g0, g1, w0, w1):
        g_sems = (g0, g1)
        w_sems = (w0, w1)
        wid = lax.axis_index("s") * nc + lax.axis_index("c")
        tb0 = wid * tb_per_w

        def unit_coords(u):
            lb = u // tb_per_w
            tb = tb0 + lax.rem(u, tb_per_w)
            return lb * LB, tb

        def stage_and_fire(u, b):
            l0, tb = unit_coords(u)
            pltpu.sync_copy(
                idx_hbm.at[pl.ds(l0, LB), pl.ds(tb * TBW, TBW)], idx_v.at[b])
            for d in range(LB):
                pltpu.async_copy(table_hbm.at[idx_v.at[b, d]],
                                 rows_v.at[b, d], g_sems[b])

        def wait_gathers(b):
            for d in range(LB):
                pltpu.make_async_copy(table_hbm.at[idx_v.at[b, d]],
                                      rows_v.at[b, d], g_sems[b]).wait()

        def transpose_unit(b):
            lane = lax.iota(jnp.int32, 16)

            for d in range(LB):
                rows2d = rows_v.at[b, d]
                tloc = t_v.at[b, d]

                @plsc.parallel_loop(0, EMB, unroll=2)
                def c_body(c):
                    col = lax.bitwise_and(c + lane, EMB - 1)
                    s_idx = lax.shift_right_logical(col, 3)
                    i_idx = lax.bitwise_and(col, 7)
                    for g in range(8):
                        row = g * 16 + lane
                        v = plsc.load_gather(rows2d, [row, col])
                        plsc.store_scatter(tloc, [s_idx, i_idx, row], v)

        def wait_write(b):
            pltpu.make_async_copy(
                t_v.at[b], out_hbm.at[pl.ds(0, LB), :, tb0],
                w_sems[b]).wait()

        stage_and_fire(0, 0)

        def body(u, carry):
            b = lax.rem(u, 2)

            nxt = jnp.minimum(u + 1, n_units - 1)

            @pl.when(b == 0)
            def _():
                stage_and_fire(nxt, 1)
                wait_gathers(0)

                @pl.when(u >= 2)
                def _():
                    wait_write(0)

                transpose_unit(0)
                l0, tb = unit_coords(u)
                pltpu.async_copy(
                    t_v.at[0],
                    out_hbm.at[pl.ds(l0, LB), :, tb], w0)

            @pl.when(b == 1)
            def _():
                stage_and_fire(nxt, 0)
                wait_gathers(1)

                @pl.when(u >= 2)
                def _():
                    wait_write(1)

                transpose_unit(1)
                l0, tb = unit_coords(u)
                pltpu.async_copy(
                    t_v.at[1],
                    out_hbm.at[pl.ds(l0, LB), :, tb], w1)

            return carry

        lax.fori_loop(0, n_units, body, 0)

        wait_write(0)
        wait_write(1)
        wait_gathers(0)

    return gather_kernel


def _build_table_conv():
    info = plsc.get_sparse_core_info()
    nc, ns = info.num_cores, info.num_subcores
    nw = nc * ns
    n_ct = 7813

    mesh = plsc.VectorSubcoreMesh(core_axis_name="c", subcore_axis_name="s")

    @functools.partial(
        pl.kernel,
        mesh=mesh,
        compiler_params=pltpu.CompilerParams(use_tc_tiling_on_sc=True,
                                             needs_layout_passes=False,
                                             disable_bounds_checks=True),
        out_type=jax.ShapeDtypeStruct((250016, 128), jnp.float32),
        scratch_types=[
            pltpu.VMEM((2, 32, 128), jnp.float32),
            pltpu.VMEM((2, 32, 128), jnp.float32),
            pltpu.SemaphoreType.DMA,
            pltpu.SemaphoreType.DMA,
            pltpu.SemaphoreType.DMA,
            pltpu.SemaphoreType.DMA,
        ],
    )
    def conv_kernel(tt_hbm, im_hbm, src_v, dst_v, i0, i1, o0, o1):
        i_sems = (i0, i1)
        o_sems = (o0, o1)
        wid = lax.axis_index("s") * nc + lax.axis_index("c")
        n_u = (n_ct - wid + nw - 1) // nw
        lane = lax.iota(jnp.int32, 16)

        def fetch(u, b):
            ct = wid + u * nw
            pltpu.async_copy(tt_hbm.at[:, pl.ds(ct * 128, 128)],
                             src_v.at[b], i_sems[b])

        def wait_fetch(b):
            pltpu.make_async_copy(tt_hbm.at[:, pl.ds(0, 128)],
                                  src_v.at[b], i_sems[b]).wait()

        def wait_out(b):
            pltpu.make_async_copy(dst_v.at[b], im_hbm.at[pl.ds(0, 32)],
                                  o_sems[b]).wait()

        def transpose(b):
            @plsc.parallel_loop(0, 32, unroll=2)
            def c_body(c):
                col = lax.bitwise_and(c + lane, 31)
                for g in range(8):
                    vbase = g * 16 + lane
                    v = plsc.load_gather(src_v.at[b], [col, vbase])
                    plsc.store_scatter(
                        dst_v.at[b],
                        [lax.shift_right_logical(vbase, 2),
                         lax.bitwise_and(vbase, 3) * 32 + col],
                        v)

        fetch(0, 0)

        def body(u, carry):
            b = lax.rem(u, 2)
            nxt = jnp.minimum(u + 1, n_u - 1)

            @pl.when(b == 0)
            def _():
                fetch(nxt, 1)
                wait_fetch(0)

                @pl.when(u >= 2)
                def _():
                    wait_out(0)

                transpose(0)
                ct = wid + u * nw
                pltpu.async_copy(dst_v.at[0],
                                 im_hbm.at[pl.ds(ct * 32, 32)], o0)

            @pl.when(b == 1)
            def _():
                fetch(nxt, 0)
                wait_fetch(1)

                @pl.when(u >= 2)
                def _():
                    wait_out(1)

                transpose(1)
                ct = wid + u * nw
                pltpu.async_copy(dst_v.at[1],
                                 im_hbm.at[pl.ds(ct * 32, 32)], o1)

            return carry

        lax.fori_loop(0, n_u, body, 0)
        wait_out(0)
        wait_out(1)

        @pl.when(lax.rem(n_u, 2) == 0)
        def _():
            wait_fetch(0)

        @pl.when(lax.rem(n_u, 2) == 1)
        def _():
            wait_fetch(1)

    return conv_kernel


def kernel(prev_actions, table):
    idxT = jnp.transpose(prev_actions)
    im = _build_table_conv()(jnp.transpose(table))
    w = _build_gather()(im.reshape(1000064, 32), idxT)
    out = w.transpose(0, 1, 3, 2, 4).reshape(L, EMB, B).transpose(2, 0, 1)
    return out
